# baseline (speedup 1.0000x reference)
import functools

import jax
import jax.numpy as jnp
from jax import lax
from jax.experimental import pallas as pl
from jax.experimental.pallas import tpu as pltpu
from jax.experimental.pallas import tpu_sc as plsc

N = 10000
E = 320000
IN_C = 128
HID = 256
OUT_C = 128

NC = 2
NS = 16
L = 16
NW = NC * NS

G = 64
NBLKP = 5120
EPAD = NBLKP * G
RPW = NBLKP // NW
CH = 4
NRING = 4
NPAD = 10240
ROWS_PER_TILE = NPAD // NS

PROWS = NPAD // 128
WIN = PROWS // NS
DEGR = EPAD // 128 // NS
DSTG = PROWS * NS

_mesh = plsc.VectorSubcoreMesh(core_axis_name="c", subcore_axis_name="s")


@functools.partial(
    pl.kernel,
    out_type=[jax.ShapeDtypeStruct((NPAD, IN_C), jnp.float32),
              jax.ShapeDtypeStruct((NPAD, IN_C), jnp.float32),
              jax.ShapeDtypeStruct((PROWS, 128), jnp.float32)],
    mesh=_mesh,
    scratch_types=[
        pltpu.VMEM((2, CH, G), jnp.int32),
        pltpu.VMEM((3, CH, G), jnp.int32),
        pltpu.VMEM((2, CH, G), jnp.float32),
        pltpu.VMEM((NRING * G, IN_C), jnp.float32),
        pltpu.VMEM((PROWS, 128), jnp.float32),
        pltpu.VMEM_SHARED((NPAD, IN_C), jnp.float32),
        pltpu.SemaphoreType.DMA,
        pltpu.SemaphoreType.DMA,
        pltpu.SemaphoreType.DMA,
        pltpu.SemaphoreType.DMA,
        pltpu.SemaphoreType.DMA,
        pltpu.SemaphoreType.DMA,
        pltpu.SemaphoreType.DMA,
        pltpu.SemaphoreType.DMA,
        pltpu.SemaphoreType.DMA,
    ],
    compiler_params=pltpu.CompilerParams(needs_layout_passes=False),
)
def _agg_kernel(src2d_hbm, dst2d_hbm, w2d_hbm, dstf_hbm, wf_hbm, x_hbm,
                out0_hbm, out1_hbm, dinv_hbm,
                se_v, de_v, we_v, rbuf_v, dinv_v, agg_sh,
                gsem0, gsem1, gsem2, gsem3,
                ssem0, ssem1, ssem2, ssem3, csem):
    cid = lax.axis_index("c")
    sid = lax.axis_index("s")
    wid = sid * NC + cid
    base = wid * RPW

    zeros = jnp.zeros((L,), jnp.float32)
    CPR = IN_C // L

    def za(i, _):
        dinv_v[i // CPR, pl.ds(pl.multiple_of((i % CPR) * L, L), L)] = zeros
        return ()

    lax.fori_loop(0, PROWS * CPR, za, ())

    for h in range(2):
        half = DEGR // 2
        pltpu.sync_copy(dstf_hbm.at[pl.ds(sid * DEGR + h * half, half)],
                        rbuf_v.at[pl.ds(0, half)])
        pltpu.sync_copy(wf_hbm.at[pl.ds(sid * DEGR + h * half, half)],
                        rbuf_v.at[pl.ds(half, half)])

        def degbody(i, _):
            for c in range(CPR):
                off = pl.multiple_of(c * L, L)
                d = rbuf_v[i, pl.ds(off, L)].astype(jnp.int32)
                wv = rbuf_v[half + i, pl.ds(off, L)]
                row = lax.shift_right_logical(d, 7)
                col = jnp.bitwise_and(d, 127)
                plsc.addupdate_scatter(dinv_v, [row, col], wv)
            return ()

        lax.fori_loop(0, half, degbody, ())

    pltpu.sync_copy(dinv_v, agg_sh.at[pl.ds(sid * PROWS, PROWS)])
    plsc.subcore_barrier()

    for t in range(NS):
        pltpu.async_copy(agg_sh.at[pl.ds(t * PROWS + sid * WIN, WIN)],
                         rbuf_v.at[pl.ds(t * WIN, WIN)], csem)
    for t in range(NS):
        pltpu.make_async_copy(agg_sh.at[pl.ds(t * PROWS + sid * WIN, WIN)],
                              rbuf_v.at[pl.ds(t * WIN, WIN)], csem).wait()

    def rsqbody(i, _):
        r = i // CPR
        off = pl.multiple_of((i % CPR) * L, L)
        acc = rbuf_v[r, pl.ds(off, L)]
        for t in range(1, NS):
            acc = acc + rbuf_v[t * WIN + r, pl.ds(off, L)]
        xdeg = acc + 1.0
        u = plsc.bitcast(xdeg, jnp.int32)
        u = 0x5F3759DF - lax.shift_right_logical(u, 1)
        rv = plsc.bitcast(u, jnp.float32)
        for _it in range(3):
            rv = rv * (1.5 - 0.5 * xdeg * rv * rv)
        rbuf_v[NS * WIN + r, pl.ds(off, L)] = rv
        return ()

    lax.fori_loop(0, WIN * CPR, rsqbody, ())

    pltpu.sync_copy(rbuf_v.at[pl.ds(NS * WIN, WIN)],
                    agg_sh.at[pl.ds(DSTG + sid * WIN, WIN)])
    plsc.subcore_barrier()
    pltpu.sync_copy(agg_sh.at[pl.ds(DSTG, PROWS)], dinv_v)

    @pl.when((cid == 0) & (sid == 0))
    def _():
        pltpu.sync_copy(agg_sh.at[pl.ds(DSTG, PROWS)], dinv_hbm)

    plsc.subcore_barrier()

    def zb(i, _):
        rem = i % (G * CPR)
        rbuf_v[(i // (G * CPR)) * G + rem // CPR,
               pl.ds(pl.multiple_of((rem % CPR) * L, L), L)] = zeros
        return ()

    lax.fori_loop(0, NRING * G * CPR, zb, ())
    for p in range(ROWS_PER_TILE // G):
        pltpu.sync_copy(rbuf_v.at[pl.ds((p % NRING) * G, G)],
                        agg_sh.at[pl.ds(sid * ROWS_PER_TILE + p * G, G)])
    plsc.subcore_barrier()

    gsems = (gsem0, gsem1, gsem2, gsem3)
    ssems = (ssem0, ssem1, ssem2, ssem3)

    def _stage_parts(c):
        rows = pl.ds(base + c * CH, CH)
        return ((src2d_hbm.at[rows], se_v.at[lax.rem(c, 2)]),
                (dst2d_hbm.at[rows], de_v.at[lax.rem(c, 3)]),
                (w2d_hbm.at[rows], we_v.at[lax.rem(c, 2)]))

    def _issue_stage(c):
        for s, d in _stage_parts(c):
            pltpu.async_copy(s, d, csem)

    def _wait_stage(c):
        for s, d in _stage_parts(c):
            pltpu.make_async_copy(s, d, csem).wait()

    def _se_row(b):
        return se_v.at[lax.rem(b // CH, 2), lax.rem(b, CH)]

    def _de_row(b):
        return de_v.at[lax.rem(b // CH, 3), lax.rem(b, CH)]

    def _ring(j):
        return rbuf_v.at[pl.ds(j * G, G)]

    def _issue_gather(b, j):
        pltpu.async_copy(x_hbm.at[_se_row(b)], _ring(j), gsems[j])

    def _wait_gather(b, j):
        pltpu.make_async_copy(x_hbm.at[_se_row(b)], _ring(j),
                              gsems[j]).wait()

    def _issue_scatter(b, j):
        pltpu.async_copy(_ring(j), agg_sh.at[_de_row(b)], ssems[j],
                         add=True)

    def _wait_scatter(b, j):
        pltpu.make_async_copy(_ring(j), agg_sh.at[_de_row(b)],
                              ssems[j]).wait()

    _issue_stage(0)
    _wait_stage(0)
    _issue_gather(0, 0)
    _issue_gather(1, 1)

    def _do_block(b, j):
        _wait_gather(b, j)

        @pl.when(b >= 2)
        def _():
            _wait_scatter(b - 2, (j + 2) % NRING)

        @pl.when((lax.rem(b + 2, CH) == 0) & (b + 2 < RPW))
        def _():
            _wait_stage((b + 2) // CH)

        @pl.when(b + 2 < RPW)
        def _():
            _issue_gather(b + 2, (j + 2) % NRING)

        @pl.when((lax.rem(b, CH) == 0) & (b + CH < RPW))
        def _():
            _issue_stage(b // CH + 1)

        def grp(i, _):
            off = pl.multiple_of(i * L, L)
            p = lax.rem(b // CH, 2)
            r = lax.rem(b, CH)
            si = se_v[p, r, pl.ds(off, L)]
            srow = lax.shift_right_logical(si, 7)
            scol = jnp.bitwise_and(si, 127)
            dsv = plsc.load_gather(dinv_v, [srow, scol])
            nv = dsv * we_v[p, r, pl.ds(off, L)]
            for jj in range(L):
                ne = nv[jj]
                row = j * G + off + jj
                for c in range(CPR):
                    rbuf_v[row, pl.ds(c * L, L)] = (
                        rbuf_v[row, pl.ds(c * L, L)] * ne)
            return ()

        lax.fori_loop(0, G // L, grp, ())

        _issue_scatter(b, j)

    def blk_body(k, _):
        for j in range(NRING):
            _do_block(NRING * k + j, j)
        return ()

    lax.fori_loop(0, RPW // NRING, blk_body, ())

    _wait_scatter(RPW - 2, (RPW - 2) % NRING)
    _wait_scatter(RPW - 1, (RPW - 1) % NRING)

    plsc.subcore_barrier()

    @pl.when(cid == 0)
    def _():
        pltpu.sync_copy(
            agg_sh.at[pl.ds(sid * ROWS_PER_TILE, ROWS_PER_TILE)],
            out0_hbm.at[pl.ds(sid * ROWS_PER_TILE, ROWS_PER_TILE)])

    @pl.when(cid == 1)
    def _():
        pltpu.sync_copy(
            agg_sh.at[pl.ds(sid * ROWS_PER_TILE, ROWS_PER_TILE)],
            out1_hbm.at[pl.ds(sid * ROWS_PER_TILE, ROWS_PER_TILE)])


def _fold_body(wz_ref, lz_ref, bz_ref, lzb_ref, wh_ref, lh_ref, bh_ref,
               lhb_ref, mz_ref, cz_ref, mh_ref, ch_ref):
    hi = lax.Precision.HIGHEST
    mz_ref[...] = jnp.dot(wz_ref[...], lz_ref[...], precision=hi)
    cz_ref[...] = jnp.dot(bz_ref[...], lz_ref[...], precision=hi) + lzb_ref[...]
    mh_ref[...] = jnp.dot(wh_ref[...], lh_ref[...], precision=hi)
    ch_ref[...] = jnp.dot(bh_ref[...], lh_ref[...], precision=hi) + lhb_ref[...]


_fold_call = pl.pallas_call(
    _fold_body,
    out_shape=[
        jax.ShapeDtypeStruct((IN_C, HID), jnp.float32),
        jax.ShapeDtypeStruct((1, HID), jnp.float32),
        jax.ShapeDtypeStruct((IN_C, HID), jnp.float32),
        jax.ShapeDtypeStruct((1, HID), jnp.float32),
    ],
)


_BN = 1000


def _dense_body(x_ref, a0_ref, a1_ref, dv_ref, mz_ref, cz_ref,
                mh_ref, ch_ref, wo_ref, bo_ref, out_ref):
    hi = lax.Precision.HIGHEST
    dv = dv_ref[...]
    a = dv * (a0_ref[...] + a1_ref[...]) + (dv * dv) * x_ref[...]
    az = jnp.dot(a, mz_ref[...], precision=hi) + cz_ref[...]
    ah = jnp.dot(a, mh_ref[...], precision=hi) + ch_ref[...]
    hn = (1.0 - jax.nn.sigmoid(az)) * jnp.tanh(ah)
    out_ref[...] = jnp.dot(hn, wo_ref[...], precision=hi) + bo_ref[...]


_dense_call = pl.pallas_call(
    _dense_body,
    grid=(N // _BN,),
    in_specs=[
        pl.BlockSpec((_BN, IN_C), lambda i: (i, 0)),
        pl.BlockSpec((_BN, IN_C), lambda i: (i, 0)),
        pl.BlockSpec((_BN, IN_C), lambda i: (i, 0)),
        pl.BlockSpec((_BN, 1), lambda i: (i, 0)),
        pl.BlockSpec((IN_C, HID), lambda i: (0, 0)),
        pl.BlockSpec((1, HID), lambda i: (0, 0)),
        pl.BlockSpec((IN_C, HID), lambda i: (0, 0)),
        pl.BlockSpec((1, HID), lambda i: (0, 0)),
        pl.BlockSpec((HID, OUT_C), lambda i: (0, 0)),
        pl.BlockSpec((1, OUT_C), lambda i: (0, 0)),
    ],
    out_specs=pl.BlockSpec((_BN, OUT_C), lambda i: (i, 0)),
    out_shape=jax.ShapeDtypeStruct((N, OUT_C), jnp.float32),
)


def kernel(x, edge_index, edge_weight, W_z, b_z, W_r, b_r, W_h, b_h,
           L_z_w, L_z_b, L_r_w, L_r_b, L_h_w, L_h_b, W_out, b_out):
    src = edge_index[0]
    dst = edge_index[1]

    pad = EPAD - E
    zi = jnp.arange(pad, dtype=jnp.int32) % N
    dstp1 = jnp.concatenate([dst, zi])
    wp1 = jnp.concatenate([edge_weight, jnp.zeros((pad,), jnp.float32)])
    srcp = jnp.concatenate([src, zi]).reshape(NBLKP, G)
    dstp = dstp1.reshape(NBLKP, G)
    wp = wp1.reshape(NBLKP, G)
    dstf = dstp1.astype(jnp.float32).reshape(EPAD // 128, 128)
    wf = wp1.reshape(EPAD // 128, 128)

    agg0, agg1, dinv_tab = _agg_kernel(srcp, dstp, wp, dstf, wf, x)
    dinv_col = dinv_tab.reshape(NPAD)[:N].reshape(N, 1)

    mz, cz, mh, ch = _fold_call(W_z, L_z_w[:HID], b_z.reshape(1, HID),
                                L_z_b.reshape(1, HID), W_h, L_h_w[:HID],
                                b_h.reshape(1, HID), L_h_b.reshape(1, HID))

    return _dense_call(x, agg0, agg1, dinv_col, mz, cz, mh, ch,
                       W_out, b_out.reshape(1, OUT_C))


# speedup vs baseline: 1.0544x; 1.0544x over previous
import functools

import jax
import jax.numpy as jnp
from jax import lax
from jax.experimental import pallas as pl
from jax.experimental.pallas import tpu as pltpu
from jax.experimental.pallas import tpu_sc as plsc

N = 10000
E = 320000
IN_C = 128
HID = 256
OUT_C = 128

NC = 2
NS = 16
L = 16
NW = NC * NS

EPW = E // NW
G = 80
NBLKP = 4096
EPAD = NBLKP * G
RPW = NBLKP // NW
CH = 8
NRING = 4
NPAD = 10240
ROWS_PER_TILE = NPAD // NS

_mesh = plsc.VectorSubcoreMesh(core_axis_name="c", subcore_axis_name="s")


@functools.partial(
    pl.kernel,
    out_type=jax.ShapeDtypeStruct((NW, N), jnp.float32),
    mesh=_mesh,
    scratch_types=[
        pltpu.VMEM((N,), jnp.float32),
        pltpu.VMEM((EPW,), jnp.int32),
        pltpu.VMEM((EPW,), jnp.float32),
    ],
    compiler_params=pltpu.CompilerParams(needs_layout_passes=False),
)
def _deg_kernel(dst_hbm, w_hbm, out_hbm, deg_v, dst_v, w_v):
    cid = lax.axis_index("c")
    sid = lax.axis_index("s")
    wid = sid * NC + cid
    base = wid * EPW

    zeros = jnp.zeros((L,), jnp.float32)

    def zbody(i, _):
        deg_v[pl.ds(pl.multiple_of(i * L, L), L)] = zeros
        return ()

    lax.fori_loop(0, N // L, zbody, ())

    pltpu.sync_copy(dst_hbm.at[pl.ds(base, EPW)], dst_v)
    pltpu.sync_copy(w_hbm.at[pl.ds(base, EPW)], w_v)

    def body(i, _):
        off = pl.multiple_of(i * L, L)
        idx = dst_v[pl.ds(off, L)]
        vals = w_v[pl.ds(off, L)]
        plsc.addupdate_scatter(deg_v, [idx], vals)
        return ()

    lax.fori_loop(0, EPW // L, body, ())

    pltpu.sync_copy(deg_v, out_hbm.at[wid])


def _dinv_body(pdeg_ref, x_ref, wz_ref, lz_ref, bz_ref, lzb_ref, wh_ref,
               lh_ref, bh_ref, lhb_ref, dinv_ref, self_ref, y_ref, mz_ref,
               cz_ref, mh_ref, ch_ref):
    deg = jnp.sum(pdeg_ref[...], axis=0, keepdims=True)
    dinv = lax.rsqrt(deg + 1.0)
    dinv_ref[...] = dinv
    self_ref[...] = dinv * dinv
    y_ref[...] = dinv.reshape(N, 1) * x_ref[...]
    hi = lax.Precision.HIGHEST
    mz_ref[...] = jnp.dot(wz_ref[...], lz_ref[...], precision=hi)
    cz_ref[...] = jnp.dot(bz_ref[...], lz_ref[...], precision=hi) + lzb_ref[...]
    mh_ref[...] = jnp.dot(wh_ref[...], lh_ref[...], precision=hi)
    ch_ref[...] = jnp.dot(bh_ref[...], lh_ref[...], precision=hi) + lhb_ref[...]


_dinv_call = pl.pallas_call(
    _dinv_body,
    out_shape=[
        jax.ShapeDtypeStruct((1, N), jnp.float32),
        jax.ShapeDtypeStruct((1, N), jnp.float32),
        jax.ShapeDtypeStruct((N, IN_C), jnp.float32),
        jax.ShapeDtypeStruct((IN_C, HID), jnp.float32),
        jax.ShapeDtypeStruct((1, HID), jnp.float32),
        jax.ShapeDtypeStruct((IN_C, HID), jnp.float32),
        jax.ShapeDtypeStruct((1, HID), jnp.float32),
    ],
)


@functools.partial(
    pl.kernel,
    out_type=[jax.ShapeDtypeStruct((NPAD, IN_C), jnp.float32),
              jax.ShapeDtypeStruct((NPAD, IN_C), jnp.float32)],
    mesh=_mesh,
    scratch_types=[
        pltpu.VMEM((2, CH, G), jnp.int32),
        pltpu.VMEM((3, CH, G), jnp.int32),
        pltpu.VMEM((2, CH, G), jnp.float32),
        pltpu.VMEM((NRING, G, IN_C), jnp.float32),
        pltpu.VMEM_SHARED((NPAD, IN_C), jnp.float32),
        pltpu.SemaphoreType.DMA,
        pltpu.SemaphoreType.DMA,
        pltpu.SemaphoreType.DMA,
        pltpu.SemaphoreType.DMA,
        pltpu.SemaphoreType.DMA,
        pltpu.SemaphoreType.DMA,
        pltpu.SemaphoreType.DMA,
        pltpu.SemaphoreType.DMA,
        pltpu.SemaphoreType.DMA,
    ],
    compiler_params=pltpu.CompilerParams(needs_layout_passes=False),
)
def _agg_kernel(src2d_hbm, dst2d_hbm, w2d_hbm, y_hbm,
                out0_hbm, out1_hbm,
                se_v, de_v, we_v, rbuf_v, agg_sh,
                gsem0, gsem1, gsem2, gsem3,
                ssem0, ssem1, ssem2, ssem3, csem):
    cid = lax.axis_index("c")
    sid = lax.axis_index("s")
    wid = sid * NC + cid
    base = wid * RPW

    def _stage_parts(c):
        rows = pl.ds(base + c * CH, CH)
        return ((src2d_hbm.at[rows], se_v.at[lax.rem(c, 2)]),
                (dst2d_hbm.at[rows], de_v.at[lax.rem(c, 3)]),
                (w2d_hbm.at[rows], we_v.at[lax.rem(c, 2)]))

    def _issue_stage(c):
        for s, d in _stage_parts(c):
            pltpu.async_copy(s, d, csem)

    def _wait_stage(c):
        for s, d in _stage_parts(c):
            pltpu.make_async_copy(s, d, csem).wait()

    def _stage_chunk(c):
        _issue_stage(c)
        _wait_stage(c)

    zeros = jnp.zeros((L,), jnp.float32)

    def zbody(i, _):
        jb = i // (G * IN_C // L)
        rem = i % (G * IN_C // L)
        r = rem // (IN_C // L)
        coff = pl.multiple_of((rem % (IN_C // L)) * L, L)
        rbuf_v[jb, r, pl.ds(coff, L)] = zeros
        return ()

    lax.fori_loop(0, NRING * G * (IN_C // L), zbody, ())
    for p in range(ROWS_PER_TILE // G):
        pltpu.sync_copy(rbuf_v.at[p % NRING],
                        agg_sh.at[pl.ds(sid * ROWS_PER_TILE + p * G, G)])
    plsc.subcore_barrier()

    gsems = (gsem0, gsem1, gsem2, gsem3)
    ssems = (ssem0, ssem1, ssem2, ssem3)

    def _se_row(b):
        return se_v.at[lax.rem(b // CH, 2), lax.rem(b, CH)]

    def _de_row(b):
        return de_v.at[lax.rem(b // CH, 3), lax.rem(b, CH)]

    def _issue_gather(b, j):
        pltpu.async_copy(y_hbm.at[_se_row(b)], rbuf_v.at[j], gsems[j])

    def _wait_gather(b, j):
        pltpu.make_async_copy(y_hbm.at[_se_row(b)], rbuf_v.at[j],
                              gsems[j]).wait()

    def _issue_scatter(b, j):
        pltpu.async_copy(rbuf_v.at[j], agg_sh.at[_de_row(b)], ssems[j],
                         add=True)

    def _wait_scatter(b, j):
        pltpu.make_async_copy(rbuf_v.at[j], agg_sh.at[_de_row(b)],
                              ssems[j]).wait()

    _stage_chunk(0)
    _issue_gather(0, 0)
    _issue_gather(1, 1)

    def _do_block(b, j):
        _wait_gather(b, j)

        @pl.when(b >= 2)
        def _():
            _wait_scatter(b - 2, (j + 2) % NRING)

        @pl.when((lax.rem(b + 2, CH) == 0) & (b + 2 < RPW))
        def _():
            _wait_stage((b + 2) // CH)

        @pl.when(b + 2 < RPW)
        def _():
            _issue_gather(b + 2, (j + 2) % NRING)

        @pl.when((lax.rem(b, CH) == 0) & (b + CH < RPW))
        def _():
            _issue_stage(b // CH + 1)

        def grp(i, _):
            off = pl.multiple_of(i * L, L)
            p = lax.rem(b // CH, 2)
            r = lax.rem(b, CH)
            nv = we_v[p, r, pl.ds(off, L)]
            for jj in range(L):
                ne = nv[jj]
                row = off + jj
                for c in range(IN_C // L):
                    rbuf_v[j, row, pl.ds(c * L, L)] = (
                        rbuf_v[j, row, pl.ds(c * L, L)] * ne)
            return ()

        lax.fori_loop(0, G // L, grp, ())

        _issue_scatter(b, j)

    def blk_body(k, _):
        for j in range(NRING):
            _do_block(NRING * k + j, j)
        return ()

    lax.fori_loop(0, RPW // NRING, blk_body, ())

    _wait_scatter(RPW - 2, (RPW - 2) % NRING)
    _wait_scatter(RPW - 1, (RPW - 1) % NRING)

    plsc.subcore_barrier()

    @pl.when(cid == 0)
    def _():
        pltpu.sync_copy(
            agg_sh.at[pl.ds(sid * ROWS_PER_TILE, ROWS_PER_TILE)],
            out0_hbm.at[pl.ds(sid * ROWS_PER_TILE, ROWS_PER_TILE)])

    @pl.when(cid == 1)
    def _():
        pltpu.sync_copy(
            agg_sh.at[pl.ds(sid * ROWS_PER_TILE, ROWS_PER_TILE)],
            out1_hbm.at[pl.ds(sid * ROWS_PER_TILE, ROWS_PER_TILE)])


_BN = 1000


def _dense_body(x_ref, a0_ref, a1_ref, dv_ref, st_ref, mz_ref, cz_ref,
                mh_ref, ch_ref, wo_ref, bo_ref, out_ref):
    hi = lax.Precision.HIGHEST
    a = dv_ref[...] * (a0_ref[...] + a1_ref[...]) + st_ref[...] * x_ref[...]
    az = jnp.dot(a, mz_ref[...], precision=hi) + cz_ref[...]
    ah = jnp.dot(a, mh_ref[...], precision=hi) + ch_ref[...]
    hn = (1.0 - jax.nn.sigmoid(az)) * jnp.tanh(ah)
    out_ref[...] = jnp.dot(hn, wo_ref[...], precision=hi) + bo_ref[...]


_dense_call = pl.pallas_call(
    _dense_body,
    grid=(N // _BN,),
    in_specs=[
        pl.BlockSpec((_BN, IN_C), lambda i: (i, 0)),
        pl.BlockSpec((_BN, IN_C), lambda i: (i, 0)),
        pl.BlockSpec((_BN, IN_C), lambda i: (i, 0)),
        pl.BlockSpec((_BN, 1), lambda i: (i, 0)),
        pl.BlockSpec((_BN, 1), lambda i: (i, 0)),
        pl.BlockSpec((IN_C, HID), lambda i: (0, 0)),
        pl.BlockSpec((1, HID), lambda i: (0, 0)),
        pl.BlockSpec((IN_C, HID), lambda i: (0, 0)),
        pl.BlockSpec((1, HID), lambda i: (0, 0)),
        pl.BlockSpec((HID, OUT_C), lambda i: (0, 0)),
        pl.BlockSpec((1, OUT_C), lambda i: (0, 0)),
    ],
    out_specs=pl.BlockSpec((_BN, OUT_C), lambda i: (i, 0)),
    out_shape=jax.ShapeDtypeStruct((N, OUT_C), jnp.float32),
)


def kernel(x, edge_index, edge_weight, W_z, b_z, W_r, b_r, W_h, b_h,
           L_z_w, L_z_b, L_r_w, L_r_b, L_h_w, L_h_b, W_out, b_out):
    src = edge_index[0]
    dst = edge_index[1]

    pdeg = _deg_kernel(dst, edge_weight)
    dinv_row, selfterm_row, y, mz, cz, mh, ch = _dinv_call(
        pdeg, x, W_z, L_z_w[:HID], b_z.reshape(1, HID),
        L_z_b.reshape(1, HID), W_h, L_h_w[:HID], b_h.reshape(1, HID),
        L_h_b.reshape(1, HID))
    dinv_col = dinv_row.reshape(N, 1)
    selfterm = selfterm_row.reshape(N, 1)

    pad = EPAD - E
    zi = jnp.arange(pad, dtype=jnp.int32) % N
    srcp = jnp.concatenate([src, zi]).reshape(NBLKP, G)
    dstp = jnp.concatenate([dst, zi]).reshape(NBLKP, G)
    wp = jnp.concatenate([edge_weight,
                          jnp.zeros((pad,), jnp.float32)]).reshape(NBLKP, G)

    agg0, agg1 = _agg_kernel(srcp, dstp, wp, y)

    return _dense_call(x, agg0, agg1, dinv_col, selfterm, mz, cz, mh,
                       ch, W_out, b_out.reshape(1, OUT_C))
